# baseline (speedup 1.0000x reference)
import functools

import jax
import jax.numpy as jnp
import numpy as np
from jax import lax
from jax.experimental import pallas as pl
from jax.experimental.pallas import tpu as pltpu
from jax.experimental.pallas import tpu_sc as plsc

N_NODES = 40962
FEAT = 128
HALF = FEAT // 2
NUM_NEW = 3 * N_NODES - 6
N_IDX = 2 * NUM_NEW
N_OUT64 = 2 * N_NODES + N_IDX
TOP64 = 2 * N_NODES

NC, NS = 2, 16
NW = NC * NS

IDX_ROWS = N_IDX // 128
IDX_PER_W = IDX_ROWS // NW

TOP_PER_W = (TOP64 // (8 * NW)) * 8
TOP_REM = TOP64 - TOP_PER_W * NW
COPY_CHUNK = 512
N_COPY_CHUNKS = TOP_PER_W // COPY_CHUNK

_ZBLK = 1024


def _tc_body(x_ref, a_ref, z_ref):
    z_ref[...] = jnp.dot(x_ref[...], a_ref[...],
                         precision=lax.Precision.HIGHEST,
                         preferred_element_type=jnp.float32)


def _make_avg_matrix():
    a = np.zeros((FEAT, HALF), np.float32)
    for f in range(HALF):
        a[2 * f, f] = 0.5
        a[2 * f + 1, f] = 0.5
    return jnp.asarray(a)


def _compute_z(x):
    n_blk = (N_NODES + _ZBLK - 1) // _ZBLK
    return pl.pallas_call(
        _tc_body,
        grid=(n_blk,),
        in_specs=[
            pl.BlockSpec((_ZBLK, FEAT), lambda i: (i, 0)),
            pl.BlockSpec((FEAT, HALF), lambda i: (0, 0)),
        ],
        out_specs=pl.BlockSpec((_ZBLK, HALF), lambda i: (i, 0)),
        out_shape=jax.ShapeDtypeStruct((N_NODES, HALF), jnp.float32),
    )(x, _make_avg_matrix())


GRP = 6
NGRP = IDX_PER_W // GRP
GROWS = GRP * 128


def _sc_body(x_hbm, z_hbm, idx_hbm, out_hbm, idx_v, buf_a, buf_b,
             sem_c, sem_ga, sem_gb, sem_wa, sem_wb):
    wid = lax.axis_index("s") * NC + lax.axis_index("c")
    tail = TOP_PER_W * NW

    top_dma = pltpu.async_copy(
        x_hbm.at[pl.ds(wid * TOP_PER_W, TOP_PER_W)],
        out_hbm.at[pl.ds(wid * TOP_PER_W, TOP_PER_W)], sem_c)

    @pl.when(wid == NW - 1)
    def _():
        pltpu.async_copy(x_hbm.at[pl.ds(tail, TOP_REM)],
                         out_hbm.at[pl.ds(tail, TOP_REM)], sem_c)

    pltpu.sync_copy(idx_hbm.at[pl.ds(wid * IDX_PER_W, IDX_PER_W)], idx_v)

    def run_group(i, g, buf, sem_g, sem_w):
        @pl.when(i > 0)
        def _():
            pltpu.make_async_copy(x_hbm.at[pl.ds(0, GROWS)], buf, sem_w
                                  ).wait()
        descs = [
            pltpu.async_copy(z_hbm.at[idx_v.at[g * GRP + j]],
                             buf.at[pl.ds(128 * j, 128)], sem_g)
            for j in range(GRP)
        ]
        for d in descs:
            d.wait()
        dst = TOP64 + (wid * IDX_PER_W + g * GRP) * 128
        pltpu.async_copy(buf, out_hbm.at[pl.ds(dst, GROWS)], sem_w)

    def body(i, carry):
        run_group(i, 2 * i, buf_a, sem_ga, sem_wa)
        run_group(i, 2 * i + 1, buf_b, sem_gb, sem_wb)
        return carry

    lax.fori_loop(0, NGRP // 2, body, 0)

    pltpu.make_async_copy(x_hbm.at[pl.ds(0, GROWS)], buf_a, sem_wa).wait()
    pltpu.make_async_copy(x_hbm.at[pl.ds(0, GROWS)], buf_b, sem_wb).wait()
    top_dma.wait()

    @pl.when(wid == NW - 1)
    def _():
        pltpu.make_async_copy(x_hbm.at[pl.ds(tail, TOP_REM)],
                              out_hbm.at[pl.ds(tail, TOP_REM)], sem_c).wait()


@jax.jit
def _run(x, idx32):
    z = _compute_z(x)
    x64 = x.reshape(TOP64, HALF)
    idx2d = idx32.reshape(IDX_ROWS, 128)
    mesh = plsc.VectorSubcoreMesh(core_axis_name="c", subcore_axis_name="s",
                                  num_cores=NC, num_subcores=NS)
    out64 = pl.kernel(
        _sc_body,
        out_type=jax.ShapeDtypeStruct((N_OUT64, HALF), jnp.float32),
        mesh=mesh,
        compiler_params=pltpu.CompilerParams(use_tc_tiling_on_sc=False),
        scratch_types=[
            pltpu.VMEM((IDX_PER_W, 128), jnp.int32),
            pltpu.VMEM((GROWS, HALF), jnp.float32),
            pltpu.VMEM((GROWS, HALF), jnp.float32),
            pltpu.SemaphoreType.DMA,
            pltpu.SemaphoreType.DMA,
            pltpu.SemaphoreType.DMA,
            pltpu.SemaphoreType.DMA,
            pltpu.SemaphoreType.DMA,
        ],
    )(x64, z, idx2d)
    return out64.reshape(N_NODES + NUM_NEW, FEAT)


def kernel(x, upsample_neighs_order):
    return _run(x, upsample_neighs_order.astype(jnp.int32))


# speedup vs baseline: 5.0990x; 5.0990x over previous
import functools

import jax
import jax.numpy as jnp
import numpy as np
from jax import lax
from jax.experimental import pallas as pl
from jax.experimental.pallas import tpu as pltpu
from jax.experimental.pallas import tpu_sc as plsc

N_NODES = 40962
FEAT = 128
HALF = FEAT // 2
NUM_NEW = 3 * N_NODES - 6
N_IDX = 2 * NUM_NEW
N_OUT64 = 2 * N_NODES + N_IDX
TOP64 = 2 * N_NODES

NC, NS = 2, 16
NW = NC * NS

IDX_ROWS = N_IDX // 128
IDX_PER_W = IDX_ROWS // NW

TOP_PER_W = (TOP64 // (8 * NW)) * 8
TOP_REM = TOP64 - TOP_PER_W * NW
COPY_CHUNK = 512
N_COPY_CHUNKS = TOP_PER_W // COPY_CHUNK

_ZBLK = 1024


def _tc_body(x_ref, a_ref, z_ref):
    z_ref[...] = jnp.dot(x_ref[...], a_ref[...],
                         precision=lax.Precision.HIGHEST,
                         preferred_element_type=jnp.float32)


def _make_avg_matrix():
    a = np.zeros((FEAT, HALF), np.float32)
    for f in range(HALF):
        a[2 * f, f] = 0.5
        a[2 * f + 1, f] = 0.5
    return jnp.asarray(a)


def _compute_z(x):
    n_blk = (N_NODES + _ZBLK - 1) // _ZBLK
    return pl.pallas_call(
        _tc_body,
        grid=(n_blk,),
        in_specs=[
            pl.BlockSpec((_ZBLK, FEAT), lambda i: (i, 0)),
            pl.BlockSpec((FEAT, HALF), lambda i: (0, 0)),
        ],
        out_specs=pl.BlockSpec((_ZBLK, HALF), lambda i: (i, 0)),
        out_shape=jax.ShapeDtypeStruct((N_NODES, HALF), jnp.float32),
    )(x, _make_avg_matrix())


GRP = 6
NGRP = IDX_PER_W // GRP
GROWS = GRP * 128


def _sc_body(x_hbm, z_hbm, idx_hbm, out_hbm, idx_v, buf_a, buf_b,
             sem_c, sem_ga, sem_gb, sem_wa, sem_wb):
    wid = lax.axis_index("s") * NC + lax.axis_index("c")
    tail = TOP_PER_W * NW

    for c in range(N_COPY_CHUNKS):
        base = wid * TOP_PER_W + c * COPY_CHUNK
        pltpu.sync_copy(x_hbm.at[pl.ds(base, COPY_CHUNK)],
                        buf_a.at[pl.ds(0, COPY_CHUNK)])
        pltpu.sync_copy(buf_a.at[pl.ds(0, COPY_CHUNK)],
                        out_hbm.at[pl.ds(base, COPY_CHUNK)])

    @pl.when(wid == NW - 1)
    def _():
        pltpu.sync_copy(x_hbm.at[pl.ds(tail, TOP_REM)],
                        buf_a.at[pl.ds(0, TOP_REM)])
        pltpu.sync_copy(buf_a.at[pl.ds(0, TOP_REM)],
                        out_hbm.at[pl.ds(tail, TOP_REM)])

    pltpu.sync_copy(idx_hbm.at[pl.ds(wid * IDX_PER_W, IDX_PER_W)], idx_v)

    def run_group(i, g, buf, sem_g, sem_w):
        @pl.when(i > 0)
        def _():
            pltpu.make_async_copy(x_hbm.at[pl.ds(0, GROWS)], buf, sem_w
                                  ).wait()
        descs = [
            pltpu.async_copy(z_hbm.at[idx_v.at[g * GRP + j]],
                             buf.at[pl.ds(128 * j, 128)], sem_g)
            for j in range(GRP)
        ]
        for d in descs:
            d.wait()
        dst = TOP64 + (wid * IDX_PER_W + g * GRP) * 128
        pltpu.async_copy(buf, out_hbm.at[pl.ds(dst, GROWS)], sem_w)

    def body(i, carry):
        run_group(i, 2 * i, buf_a, sem_ga, sem_wa)
        run_group(i, 2 * i + 1, buf_b, sem_gb, sem_wb)
        return carry

    lax.fori_loop(0, NGRP // 2, body, 0)

    pltpu.make_async_copy(x_hbm.at[pl.ds(0, GROWS)], buf_a, sem_wa).wait()
    pltpu.make_async_copy(x_hbm.at[pl.ds(0, GROWS)], buf_b, sem_wb).wait()


@jax.jit
def _run(x, idx32):
    z = _compute_z(x)
    x64 = x.reshape(TOP64, HALF)
    idx2d = idx32.reshape(IDX_ROWS, 128)
    mesh = plsc.VectorSubcoreMesh(core_axis_name="c", subcore_axis_name="s",
                                  num_cores=NC, num_subcores=NS)
    out64 = pl.kernel(
        _sc_body,
        out_type=jax.ShapeDtypeStruct((N_OUT64, HALF), jnp.float32),
        mesh=mesh,
        compiler_params=pltpu.CompilerParams(use_tc_tiling_on_sc=False),
        scratch_types=[
            pltpu.VMEM((IDX_PER_W, 128), jnp.int32),
            pltpu.VMEM((GROWS, HALF), jnp.float32),
            pltpu.VMEM((GROWS, HALF), jnp.float32),
            pltpu.SemaphoreType.DMA,
            pltpu.SemaphoreType.DMA,
            pltpu.SemaphoreType.DMA,
            pltpu.SemaphoreType.DMA,
            pltpu.SemaphoreType.DMA,
        ],
    )(x64, z, idx2d)
    return out64.reshape(N_NODES + NUM_NEW, FEAT)


def kernel(x, upsample_neighs_order):
    return _run(x, upsample_neighs_order.astype(jnp.int32))


# speedup vs baseline: 5.3116x; 1.0417x over previous
import functools

import jax
import jax.numpy as jnp
import numpy as np
from jax import lax
from jax.experimental import pallas as pl
from jax.experimental.pallas import tpu as pltpu
from jax.experimental.pallas import tpu_sc as plsc

N_NODES = 40962
FEAT = 128
HALF = FEAT // 2
NUM_NEW = 3 * N_NODES - 6
N_IDX = 2 * NUM_NEW
N_OUT64 = 2 * N_NODES + N_IDX
TOP64 = 2 * N_NODES

NC, NS = 2, 16
NW = NC * NS

N_GW = 24
N_CW = NW - N_GW
IDX_ROWS = N_IDX // 128
IDX_PER_W = IDX_ROWS // N_GW

TOP_PER_CW = TOP64 // N_CW
TOP_REM = TOP64 - TOP_PER_CW * N_CW
COPY_CHUNK = 640
N_COPY_CHUNKS = TOP_PER_CW // COPY_CHUNK

_ZBLK = 1024


def _tc_body(x_ref, a_ref, z_ref):
    z_ref[...] = jnp.dot(x_ref[...], a_ref[...],
                         precision=lax.Precision.HIGHEST,
                         preferred_element_type=jnp.float32)


def _make_avg_matrix():
    a = np.zeros((FEAT, HALF), np.float32)
    for f in range(HALF):
        a[2 * f, f] = 0.5
        a[2 * f + 1, f] = 0.5
    return jnp.asarray(a)


def _compute_z(x):
    n_blk = (N_NODES + _ZBLK - 1) // _ZBLK
    return pl.pallas_call(
        _tc_body,
        grid=(n_blk,),
        in_specs=[
            pl.BlockSpec((_ZBLK, FEAT), lambda i: (i, 0)),
            pl.BlockSpec((FEAT, HALF), lambda i: (0, 0)),
        ],
        out_specs=pl.BlockSpec((_ZBLK, HALF), lambda i: (i, 0)),
        out_shape=jax.ShapeDtypeStruct((N_NODES, HALF), jnp.float32),
    )(x, _make_avg_matrix())


GRP = 5
NGRP = IDX_PER_W // GRP
GROWS = GRP * 128


def _sc_body(x_hbm, z_hbm, idx_hbm, out_hbm, idx_v, buf_a, buf_b,
             sem_ga, sem_gb, sem_wa, sem_wb):
    wid = lax.axis_index("s") * NC + lax.axis_index("c")

    @pl.when(wid < N_GW)
    def _gather():
        pltpu.sync_copy(idx_hbm.at[pl.ds(wid * IDX_PER_W, IDX_PER_W)], idx_v)

        def run_group(i, g, buf, sem_g, sem_w):
            @pl.when(i > 0)
            def _():
                pltpu.make_async_copy(x_hbm.at[pl.ds(0, GROWS)], buf, sem_w
                                      ).wait()
            descs = [
                pltpu.async_copy(z_hbm.at[idx_v.at[g * GRP + j]],
                                 buf.at[pl.ds(128 * j, 128)], sem_g)
                for j in range(GRP)
            ]
            for d in descs:
                d.wait()
            dst = TOP64 + (wid * IDX_PER_W + g * GRP) * 128
            pltpu.async_copy(buf, out_hbm.at[pl.ds(dst, GROWS)], sem_w)

        def body(i, carry):
            run_group(i, 2 * i, buf_a, sem_ga, sem_wa)
            run_group(i, 2 * i + 1, buf_b, sem_gb, sem_wb)
            return carry

        lax.fori_loop(0, NGRP // 2, body, 0)

        pltpu.make_async_copy(x_hbm.at[pl.ds(0, GROWS)], buf_a, sem_wa).wait()
        pltpu.make_async_copy(x_hbm.at[pl.ds(0, GROWS)], buf_b, sem_wb).wait()

    @pl.when(wid >= N_GW)
    def _copy():
        cw = wid - N_GW
        for c in range(N_COPY_CHUNKS):
            buf, sem_in, sem_out = ((buf_a, sem_ga, sem_wa) if c % 2 == 0
                                    else (buf_b, sem_gb, sem_wb))
            base = cw * TOP_PER_CW + c * COPY_CHUNK
            if c >= 2:
                pltpu.make_async_copy(x_hbm.at[pl.ds(0, COPY_CHUNK)], buf,
                                      sem_out).wait()
            pltpu.async_copy(x_hbm.at[pl.ds(base, COPY_CHUNK)], buf,
                             sem_in).wait()
            pltpu.async_copy(buf, out_hbm.at[pl.ds(base, COPY_CHUNK)],
                             sem_out)
        pltpu.make_async_copy(x_hbm.at[pl.ds(0, COPY_CHUNK)], buf_a,
                              sem_wa).wait()
        pltpu.make_async_copy(x_hbm.at[pl.ds(0, COPY_CHUNK)], buf_b,
                              sem_wb).wait()

        @pl.when(wid == NW - 1)
        def _():
            tail = TOP_PER_CW * N_CW
            pltpu.sync_copy(x_hbm.at[pl.ds(tail, TOP_REM)],
                            buf_a.at[pl.ds(0, TOP_REM)])
            pltpu.sync_copy(buf_a.at[pl.ds(0, TOP_REM)],
                            out_hbm.at[pl.ds(tail, TOP_REM)])


@jax.jit
def _run(x, idx32):
    z = _compute_z(x)
    x64 = x.reshape(TOP64, HALF)
    idx2d = idx32.reshape(IDX_ROWS, 128)
    mesh = plsc.VectorSubcoreMesh(core_axis_name="c", subcore_axis_name="s",
                                  num_cores=NC, num_subcores=NS)
    out64 = pl.kernel(
        _sc_body,
        out_type=jax.ShapeDtypeStruct((N_OUT64, HALF), jnp.float32),
        mesh=mesh,
        compiler_params=pltpu.CompilerParams(use_tc_tiling_on_sc=False),
        scratch_types=[
            pltpu.VMEM((IDX_PER_W, 128), jnp.int32),
            pltpu.VMEM((GROWS, HALF), jnp.float32),
            pltpu.VMEM((GROWS, HALF), jnp.float32),
            pltpu.SemaphoreType.DMA,
            pltpu.SemaphoreType.DMA,
            pltpu.SemaphoreType.DMA,
            pltpu.SemaphoreType.DMA,
        ],
    )(x64, z, idx2d)
    return out64.reshape(N_NODES + NUM_NEW, FEAT)


def kernel(x, upsample_neighs_order):
    return _run(x, upsample_neighs_order.astype(jnp.int32))


# speedup vs baseline: 5.9635x; 1.1227x over previous
import functools

import jax
import jax.numpy as jnp
import numpy as np
from jax import lax
from jax.experimental import pallas as pl
from jax.experimental.pallas import tpu as pltpu
from jax.experimental.pallas import tpu_sc as plsc

N_NODES = 40962
FEAT = 128
HALF = FEAT // 2
NUM_NEW = 3 * N_NODES - 6
N_IDX = 2 * NUM_NEW
N_OUT64 = 2 * N_NODES + N_IDX
TOP64 = 2 * N_NODES

NC, NS = 2, 16
NW = NC * NS

N_GW = 24
N_CW = NW - N_GW
IDX_ROWS = N_IDX // 128
IDX_PER_W = IDX_ROWS // N_GW

TOP_PER_CW = TOP64 // N_CW
TOP_REM = TOP64 - TOP_PER_CW * N_CW
COPY_CHUNK = 640
N_COPY_CHUNKS = TOP_PER_CW // COPY_CHUNK

_ZBLK = 4096


def _tc_body(x_ref, a_ref, z_ref):
    z_ref[...] = jnp.dot(x_ref[...], a_ref[...],
                         precision=lax.Precision.HIGHEST,
                         preferred_element_type=jnp.float32)


def _make_avg_matrix():
    a = np.zeros((FEAT, HALF), np.float32)
    for f in range(HALF):
        a[2 * f, f] = 0.5
        a[2 * f + 1, f] = 0.5
    return jnp.asarray(a)


def _compute_z(x):
    n_blk = (N_NODES + _ZBLK - 1) // _ZBLK
    return pl.pallas_call(
        _tc_body,
        grid=(n_blk,),
        in_specs=[
            pl.BlockSpec((_ZBLK, FEAT), lambda i: (i, 0)),
            pl.BlockSpec((FEAT, HALF), lambda i: (0, 0)),
        ],
        out_specs=pl.BlockSpec((_ZBLK, HALF), lambda i: (i, 0)),
        out_shape=jax.ShapeDtypeStruct((N_NODES, HALF), jnp.float32),
    )(x, _make_avg_matrix())


GRP = 5
NGRP = IDX_PER_W // GRP
GROWS = GRP * 128


def _sc_body(x_hbm, z_hbm, idx_hbm, out_hbm, idx_v, buf_a, buf_b,
             sem_ga, sem_gb, sem_wa, sem_wb):
    wid = lax.axis_index("s") * NC + lax.axis_index("c")

    @pl.when(wid < N_GW)
    def _gather():
        pltpu.sync_copy(idx_hbm.at[pl.ds(wid * IDX_PER_W, IDX_PER_W)], idx_v)

        def run_group(i, g, buf, sem_g, sem_w):
            @pl.when(i > 0)
            def _():
                pltpu.make_async_copy(x_hbm.at[pl.ds(0, GROWS)], buf, sem_w
                                      ).wait()
            descs = [
                pltpu.async_copy(z_hbm.at[idx_v.at[g * GRP + j]],
                                 buf.at[pl.ds(128 * j, 128)], sem_g)
                for j in range(GRP)
            ]
            for d in descs:
                d.wait()
            dst = TOP64 + (wid * IDX_PER_W + g * GRP) * 128
            pltpu.async_copy(buf, out_hbm.at[pl.ds(dst, GROWS)], sem_w)

        def body(i, carry):
            run_group(i, 2 * i, buf_a, sem_ga, sem_wa)
            run_group(i, 2 * i + 1, buf_b, sem_gb, sem_wb)
            return carry

        lax.fori_loop(0, NGRP // 2, body, 0)

        pltpu.make_async_copy(x_hbm.at[pl.ds(0, GROWS)], buf_a, sem_wa).wait()
        pltpu.make_async_copy(x_hbm.at[pl.ds(0, GROWS)], buf_b, sem_wb).wait()

    @pl.when(wid >= N_GW)
    def _copy():
        cw = wid - N_GW
        for c in range(N_COPY_CHUNKS):
            buf, sem_in, sem_out = ((buf_a, sem_ga, sem_wa) if c % 2 == 0
                                    else (buf_b, sem_gb, sem_wb))
            base = cw * TOP_PER_CW + c * COPY_CHUNK
            if c >= 2:
                pltpu.make_async_copy(x_hbm.at[pl.ds(0, COPY_CHUNK)], buf,
                                      sem_out).wait()
            pltpu.async_copy(x_hbm.at[pl.ds(base, COPY_CHUNK)], buf,
                             sem_in).wait()
            pltpu.async_copy(buf, out_hbm.at[pl.ds(base, COPY_CHUNK)],
                             sem_out)
        pltpu.make_async_copy(x_hbm.at[pl.ds(0, COPY_CHUNK)], buf_a,
                              sem_wa).wait()
        pltpu.make_async_copy(x_hbm.at[pl.ds(0, COPY_CHUNK)], buf_b,
                              sem_wb).wait()

        @pl.when(wid == NW - 1)
        def _():
            tail = TOP_PER_CW * N_CW
            pltpu.sync_copy(x_hbm.at[pl.ds(tail, TOP_REM)],
                            buf_a.at[pl.ds(0, TOP_REM)])
            pltpu.sync_copy(buf_a.at[pl.ds(0, TOP_REM)],
                            out_hbm.at[pl.ds(tail, TOP_REM)])


@jax.jit
def _run(x, idx32):
    z = _compute_z(x)
    x64 = x.reshape(TOP64, HALF)
    idx2d = idx32.reshape(IDX_ROWS, 128)
    mesh = plsc.VectorSubcoreMesh(core_axis_name="c", subcore_axis_name="s",
                                  num_cores=NC, num_subcores=NS)
    out64 = pl.kernel(
        _sc_body,
        out_type=jax.ShapeDtypeStruct((N_OUT64, HALF), jnp.float32),
        mesh=mesh,
        compiler_params=pltpu.CompilerParams(use_tc_tiling_on_sc=False),
        scratch_types=[
            pltpu.VMEM((IDX_PER_W, 128), jnp.int32),
            pltpu.VMEM((GROWS, HALF), jnp.float32),
            pltpu.VMEM((GROWS, HALF), jnp.float32),
            pltpu.SemaphoreType.DMA,
            pltpu.SemaphoreType.DMA,
            pltpu.SemaphoreType.DMA,
            pltpu.SemaphoreType.DMA,
        ],
    )(x64, z, idx2d)
    return out64.reshape(N_NODES + NUM_NEW, FEAT)


def kernel(x, upsample_neighs_order):
    return _run(x, upsample_neighs_order.astype(jnp.int32))
